# initial kernel scaffold (unmeasured)
import jax
import jax.numpy as jnp
from jax import lax
from jax.experimental import pallas as pl
from jax.experimental.pallas import tpu as pltpu


def kernel(
    x,
):
    def body(*refs):
        pass

    out_shape = jax.ShapeDtypeStruct(..., jnp.float32)
    return pl.pallas_call(body, out_shape=out_shape)(...)


# baseline (device time: 206266 ns/iter reference)
import jax
import jax.numpy as jnp
from jax import lax
from jax.experimental import pallas as pl
from jax.experimental.pallas import tpu as pltpu

N_DEV = 16


def kernel(x):
    m_per, n = x.shape

    def body(x_ref, out_ref, send_sems, recv_sems):
        my_pos = lax.axis_index("i")
        left = lax.rem(my_pos - 1 + N_DEV, N_DEV)
        right = lax.rem(my_pos + 1, N_DEV)

        barrier_sem = pltpu.get_barrier_semaphore()
        pl.semaphore_signal(
            barrier_sem, inc=1, device_id=(left,),
            device_id_type=pl.DeviceIdType.MESH,
        )
        pl.semaphore_signal(
            barrier_sem, inc=1, device_id=(right,),
            device_id_type=pl.DeviceIdType.MESH,
        )
        pl.semaphore_wait(barrier_sem, 2)

        out_ref[pl.ds(my_pos * m_per, m_per), :] = x_ref[:, :]

        for h in range(N_DEV - 1):
            send_origin = lax.rem(my_pos - h + N_DEV, N_DEV)
            recv_origin = lax.rem(my_pos - h - 1 + N_DEV, N_DEV)
            rdma = pltpu.make_async_remote_copy(
                src_ref=out_ref.at[pl.ds(send_origin * m_per, m_per), :],
                dst_ref=out_ref.at[pl.ds(send_origin * m_per, m_per), :],
                send_sem=send_sems.at[h],
                recv_sem=recv_sems.at[h],
                device_id=(right,),
                device_id_type=pl.DeviceIdType.MESH,
            )
            rdma.start()
            rdma.wait()
            del recv_origin

    return pl.pallas_call(
        body,
        out_shape=jax.ShapeDtypeStruct((N_DEV * m_per, n), x.dtype),
        in_specs=[pl.BlockSpec(memory_space=pltpu.VMEM)],
        out_specs=pl.BlockSpec(memory_space=pltpu.VMEM),
        scratch_shapes=[
            pltpu.SemaphoreType.DMA((N_DEV - 1,)),
            pltpu.SemaphoreType.DMA((N_DEV - 1,)),
        ],
        compiler_params=pltpu.CompilerParams(collective_id=0),
    )(x)


# device time: 127083 ns/iter; 1.6231x vs baseline; 1.6231x over previous
import jax
import jax.numpy as jnp
from jax import lax
from jax.experimental import pallas as pl
from jax.experimental.pallas import tpu as pltpu

N_DEV = 16
H_RIGHT = 8
H_LEFT = 7


def kernel(x):
    m_per, n = x.shape

    def body(x_ref, out_ref, send_r, recv_r, send_l, recv_l):
        my_pos = lax.axis_index("i")
        left = lax.rem(my_pos - 1 + N_DEV, N_DEV)
        right = lax.rem(my_pos + 1, N_DEV)

        barrier_sem = pltpu.get_barrier_semaphore()
        pl.semaphore_signal(
            barrier_sem, inc=1, device_id=(left,),
            device_id_type=pl.DeviceIdType.MESH,
        )
        pl.semaphore_signal(
            barrier_sem, inc=1, device_id=(right,),
            device_id_type=pl.DeviceIdType.MESH,
        )
        pl.semaphore_wait(barrier_sem, 2)

        out_ref[pl.ds(my_pos * m_per, m_per), :] = x_ref[:, :]

        def chunk(origin):
            return out_ref.at[pl.ds(origin * m_per, m_per), :]

        for h in range(H_RIGHT):
            r_origin = lax.rem(my_pos - h + N_DEV, N_DEV)
            rdma_r = pltpu.make_async_remote_copy(
                src_ref=chunk(r_origin),
                dst_ref=chunk(r_origin),
                send_sem=send_r.at[h],
                recv_sem=recv_r.at[h],
                device_id=(right,),
                device_id_type=pl.DeviceIdType.MESH,
            )
            rdma_r.start()
            if h < H_LEFT:
                l_origin = lax.rem(my_pos + h, N_DEV)
                rdma_l = pltpu.make_async_remote_copy(
                    src_ref=chunk(l_origin),
                    dst_ref=chunk(l_origin),
                    send_sem=send_l.at[h],
                    recv_sem=recv_l.at[h],
                    device_id=(left,),
                    device_id_type=pl.DeviceIdType.MESH,
                )
                rdma_l.start()
                rdma_l.wait()
            rdma_r.wait()

    return pl.pallas_call(
        body,
        out_shape=jax.ShapeDtypeStruct((N_DEV * m_per, n), x.dtype),
        in_specs=[pl.BlockSpec(memory_space=pltpu.VMEM)],
        out_specs=pl.BlockSpec(memory_space=pltpu.VMEM),
        scratch_shapes=[
            pltpu.SemaphoreType.DMA((H_RIGHT,)),
            pltpu.SemaphoreType.DMA((H_RIGHT,)),
            pltpu.SemaphoreType.DMA((H_LEFT,)),
            pltpu.SemaphoreType.DMA((H_LEFT,)),
        ],
        compiler_params=pltpu.CompilerParams(collective_id=0),
    )(x)


# device time: 99375 ns/iter; 2.0756x vs baseline; 1.2788x over previous
import jax
import jax.numpy as jnp
from jax import lax
from jax.experimental import pallas as pl
from jax.experimental.pallas import tpu as pltpu

N_DEV = 16
N_MSG = N_DEV - 1


def kernel(x):
    m_per, n = x.shape
    m_half = m_per // 2

    def body(x_ref, out_ref, send_r, recv_r, send_l, recv_l):
        my_pos = lax.axis_index("i")
        left = lax.rem(my_pos - 1 + N_DEV, N_DEV)
        right = lax.rem(my_pos + 1, N_DEV)

        barrier_sem = pltpu.get_barrier_semaphore()
        pl.semaphore_signal(
            barrier_sem, inc=1, device_id=(left,),
            device_id_type=pl.DeviceIdType.MESH,
        )
        pl.semaphore_signal(
            barrier_sem, inc=1, device_id=(right,),
            device_id_type=pl.DeviceIdType.MESH,
        )
        pl.semaphore_wait(barrier_sem, 2)

        out_ref[pl.ds(my_pos * m_per, m_per), :] = x_ref[:, :]

        def half(origin, h):
            return out_ref.at[pl.ds(origin * m_per + h * m_half, m_half), :]

        out_r, in_r, out_l, in_l = [], [], [], []
        for k in range(N_MSG):
            j = k // 2
            hr = k % 2
            hl = 1 - hr
            o_out_r = lax.rem(my_pos - j + N_DEV, N_DEV)
            o_in_r = lax.rem(my_pos - j - 1 + N_DEV, N_DEV)
            o_out_l = lax.rem(my_pos + j, N_DEV)
            o_in_l = lax.rem(my_pos + j + 1, N_DEV)
            out_r.append(pltpu.make_async_remote_copy(
                src_ref=half(o_out_r, hr), dst_ref=half(o_out_r, hr),
                send_sem=send_r.at[k], recv_sem=recv_r.at[k],
                device_id=(right,), device_id_type=pl.DeviceIdType.MESH,
            ))
            in_r.append(pltpu.make_async_remote_copy(
                src_ref=half(o_in_r, hr), dst_ref=half(o_in_r, hr),
                send_sem=send_r.at[k], recv_sem=recv_r.at[k],
                device_id=(right,), device_id_type=pl.DeviceIdType.MESH,
            ))
            out_l.append(pltpu.make_async_remote_copy(
                src_ref=half(o_out_l, hl), dst_ref=half(o_out_l, hl),
                send_sem=send_l.at[k], recv_sem=recv_l.at[k],
                device_id=(left,), device_id_type=pl.DeviceIdType.MESH,
            ))
            in_l.append(pltpu.make_async_remote_copy(
                src_ref=half(o_in_l, hl), dst_ref=half(o_in_l, hl),
                send_sem=send_l.at[k], recv_sem=recv_l.at[k],
                device_id=(left,), device_id_type=pl.DeviceIdType.MESH,
            ))

        for k in range(N_MSG):
            if k >= 2:
                in_r[k - 2].wait_recv()
                in_l[k - 2].wait_recv()
            out_r[k].start()
            out_l[k].start()

        for k in (N_MSG - 2, N_MSG - 1):
            in_r[k].wait_recv()
            in_l[k].wait_recv()
        for k in range(N_MSG):
            out_r[k].wait_send()
            out_l[k].wait_send()

    return pl.pallas_call(
        body,
        out_shape=jax.ShapeDtypeStruct((N_DEV * m_per, n), x.dtype),
        in_specs=[pl.BlockSpec(memory_space=pltpu.VMEM)],
        out_specs=pl.BlockSpec(memory_space=pltpu.VMEM),
        scratch_shapes=[
            pltpu.SemaphoreType.DMA((N_MSG,)),
            pltpu.SemaphoreType.DMA((N_MSG,)),
            pltpu.SemaphoreType.DMA((N_MSG,)),
            pltpu.SemaphoreType.DMA((N_MSG,)),
        ],
        compiler_params=pltpu.CompilerParams(collective_id=0),
    )(x)
